# baseline (speedup 1.0000x reference)
import functools

import jax
import jax.numpy as jnp
from jax import lax
from jax.experimental import pallas as pl
from jax.experimental.pallas import tpu as pltpu
from jax.experimental.pallas import tpu_sc as plsc

_N = 50000
_E = 800000
_B = 512
_H = 64
_HH = 32
_NLAYERS = 8
_NC = 2
_NS = 16
_NW = _NC * _NS

_NP = 51200
_RPT = _NP // _NS
_N2 = 2 * _NP

_K = 128
_G = 3
_GE = _G * _K
_EP = 801024
_EG = _EP // _GE

_ND = 50048
_DC = _ND // _K

_mesh = plsc.VectorSubcoreMesh(core_axis_name="c", subcore_axis_name="s")
_sc_params = pltpu.CompilerParams(
    needs_layout_passes=False, use_tc_tiling_on_sc=False)


@functools.partial(
    pl.kernel,
    out_type=jax.ShapeDtypeStruct((_NW * _NP,), jnp.float32),
    mesh=_mesh,
    compiler_params=_sc_params,
    scratch_types=[
        pltpu.VMEM((_NP,), jnp.float32),
        pltpu.VMEM((_NP,), jnp.float32),
        pltpu.VMEM((_G, _K), jnp.int32),
        pltpu.VMEM((_G, _K), jnp.int32),
    ],
)
def _agg1_kernel(x1_hbm, src_hbm, dst_hbm, out_hbm, xv, aggv, srcv, dstv):
    c = lax.axis_index("c")
    s = lax.axis_index("s")
    wid = s * _NC + c
    pltpu.sync_copy(x1_hbm, xv)
    zero16 = jnp.zeros((16,), jnp.float32)

    def _zero(i, carry):
        aggv[pl.ds(i * 16, 16)] = zero16
        return carry

    lax.fori_loop(0, _NP // 16, _zero, 0)

    def _grp(i, carry):
        gg = wid + i * _NW

        @pl.when(gg < _EG)
        def _():
            pltpu.sync_copy(src_hbm.at[gg], srcv)
            pltpu.sync_copy(dst_hbm.at[gg], dstv)
            for j in range(_G):
                for k in range(_K // 16):
                    sidx = srcv[j, pl.ds(k * 16, 16)]
                    didx = dstv[j, pl.ds(k * 16, 16)]
                    vals = plsc.load_gather(xv, [sidx])
                    plsc.addupdate_scatter(aggv, [didx], vals)

        return carry

    lax.fori_loop(0, (_EG + _NW - 1) // _NW, _grp, 0)
    pltpu.sync_copy(aggv, out_hbm.at[pl.ds(wid * _NP, _NP)])


@functools.partial(
    pl.kernel,
    out_type=jax.ShapeDtypeStruct((_N2, _HH), jnp.float32),
    mesh=_mesh,
    compiler_params=_sc_params,
    scratch_types=[
        pltpu.VMEM((2, _GE, _HH), jnp.float32),
        pltpu.VMEM((3, _G, _K), jnp.int32),
        pltpu.VMEM((3, _G, _K), jnp.int32),
        pltpu.VMEM_SHARED((_NP + 8, _HH), jnp.float32),
        pltpu.SemaphoreType.DMA,
        pltpu.SemaphoreType.DMA,
        pltpu.SemaphoreType.DMA,
    ],
)
def _gin_agg_kernel(x_hbm, src_hbm, dst_hbm, z_hbm,
                    rows, srcv, dstv, agg_sh, gsem, isem, ssem):
    c = lax.axis_index("c")
    s = lax.axis_index("s")
    lbase = s * _RPT
    gbase = c * _NP + s * _RPT

    pltpu.sync_copy(x_hbm.at[pl.ds(gbase, _RPT)], agg_sh.at[pl.ds(lbase, _RPT)])
    plsc.subcore_barrier()

    base = x_hbm.at[pl.ds(c * _NP, _NP)]

    def _grp(i, carry):
        slot = lax.rem(i, 3)
        nslot = lax.rem(i + 1, 3)
        dslot = lax.rem(i + 1, 3)
        p = lax.rem(i, 2)
        gg = s + i * _NS

        @pl.when(gg < _EG)
        def _():
            @pl.when(i == 0)
            def _():
                pltpu.sync_copy(src_hbm.at[gg], srcv.at[0])
                pltpu.sync_copy(dst_hbm.at[gg], dstv.at[0])

            @pl.when(i > 0)
            def _():
                pltpu.make_async_copy(src_hbm.at[gg], srcv.at[slot],
                                      isem).wait()
                pltpu.make_async_copy(dst_hbm.at[gg], dstv.at[slot],
                                      isem).wait()

            @pl.when(i > 1)
            def _():
                for j in range(_G):
                    pltpu.make_async_copy(
                        rows.at[p, pl.ds(j * _K, _K)],
                        agg_sh.at[dstv.at[dslot, j]], ssem).wait()

            gn = gg + _NS

            @pl.when(gn < _EG)
            def _():
                pltpu.async_copy(src_hbm.at[gn], srcv.at[nslot], isem)
                pltpu.async_copy(dst_hbm.at[gn], dstv.at[nslot], isem)

            gs = [
                pltpu.async_copy(base.at[srcv.at[slot, j]],
                                 rows.at[p, pl.ds(j * _K, _K)], gsem)
                for j in range(_G)
            ]
            for j in range(_G):
                gs[j].wait()
                pltpu.async_copy(rows.at[p, pl.ds(j * _K, _K)],
                                 agg_sh.at[dstv.at[slot, j]], ssem, add=True)

        return carry

    lax.fori_loop(0, (_EG + _NS - 1) // _NS, _grp, 0)

    ni = (_EG - 1 - s) // _NS + 1

    def _drain(d):
        @pl.when(d >= 0)
        def _():
            pd = lax.rem(d, 2)
            sd = lax.rem(d, 3)
            for j in range(_G):
                pltpu.make_async_copy(rows.at[pd, pl.ds(j * _K, _K)],
                                      agg_sh.at[dstv.at[sd, j]], ssem).wait()

    _drain(ni - 2)
    _drain(ni - 1)
    plsc.subcore_barrier()

    pltpu.sync_copy(agg_sh.at[pl.ds(lbase, _RPT)], z_hbm.at[pl.ds(gbase, _RPT)])


_pool_out = tuple(
    jax.ShapeDtypeStruct((_NC, _B, _HH), jnp.float32)
    for _ in range(2 * _NLAYERS + 1)
)
_PSH = 528


@functools.partial(
    pl.kernel,
    out_type=_pool_out,
    mesh=_mesh,
    compiler_params=_sc_params,
    scratch_types=[
        pltpu.VMEM((2 * _NLAYERS, _K, _HH), jnp.float32),
        pltpu.VMEM((_K, _HH), jnp.float32),
        pltpu.VMEM((1, _K), jnp.int32),
        pltpu.SemaphoreType.DMA,
    ]
    + [pltpu.VMEM_SHARED((_PSH, _HH), jnp.float32)
       for _ in range(2 * _NLAYERS + 1)],
)
def _pool_kernel(h0, h1, h2, h3, h4, h5, h6, h7, batch_hbm, *rest):
    outs = rest[:2 * _NLAYERS + 1]
    rows16, ones, bidx, sem = rest[2 * _NLAYERS + 1:2 * _NLAYERS + 5]
    shs = rest[2 * _NLAYERS + 5:]
    c = lax.axis_index("c")
    s = lax.axis_index("s")
    wid = s * _NC + c
    hs = (h0, h1, h2, h3, h4, h5, h6, h7)
    zero16 = jnp.zeros((16,), jnp.float32)
    one16 = jnp.ones((16,), jnp.float32)

    def _init(i, carry):
        for k in range(_HH // 16):
            ones[i, pl.ds(k * 16, 16)] = one16
            rows16[0, i, pl.ds(k * 16, 16)] = zero16
        return carry

    lax.fori_loop(0, _K, _init, 0)
    for sh in shs:
        pltpu.sync_copy(rows16.at[0, pl.ds(0, 33)], sh.at[pl.ds(s * 33, 33)])
    plsc.subcore_barrier()

    def _chunk(i, carry):
        cc = wid + i * _NW

        @pl.when(cc < _DC)
        def _():
            pltpu.sync_copy(batch_hbm.at[cc], bidx)
            cps = []
            for l in range(_NLAYERS):
                for half in range(2):
                    cps.append(pltpu.async_copy(
                        hs[l].at[pl.ds(half * _NP + cc * _K, _K)],
                        rows16.at[2 * l + half], sem))
            for q, cp in enumerate(cps):
                cp.wait()
                pltpu.sync_copy(rows16.at[q], shs[q].at[bidx.at[0]], add=True)
            pltpu.sync_copy(ones, shs[2 * _NLAYERS].at[bidx.at[0]], add=True)

        return carry

    lax.fori_loop(0, (_DC + _NW - 1) // _NW, _chunk, 0)
    plsc.subcore_barrier()
    for q in range(2 * _NLAYERS + 1):
        pltpu.sync_copy(shs[q].at[pl.ds(s * 32, 32)],
                        outs[q].at[c, pl.ds(s * 32, 32)])


_BLK = 2048
_NBLK = _NP // _BLK


_PBLK = 8
_PROWS = 12800 // _PBLK


def _store_half(o_ref, h):
    half = pl.program_id(0) // _NBLK

    @pl.when(half == 0)
    def _():
        o_ref[...] = h[:, :_HH]

    @pl.when(half == 1)
    def _():
        o_ref[...] = h[:, _HH:]


def _mlp_body(zl_ref, zh_ref, xl_ref, xh_ref, eps_ref,
              w1, b1, s1, be1, w2, b2, s2, be2, o_ref):
    half = pl.program_id(0) // _PBLK
    e = eps_ref[0, 0]
    zl = zl_ref[...] + e * xl_ref[...]
    zh = zh_ref[...] + e * xh_ref[...]
    outs = []
    for q in range(4):
        z = jnp.concatenate(
            [zl[:, q * _HH:(q + 1) * _HH], zh[:, q * _HH:(q + 1) * _HH]],
            axis=1)
        t = jnp.maximum(
            jnp.dot(z, w1[...], preferred_element_type=jnp.float32)
            + b1[...], 0.0)
        t = t * s1[...] + be1[...]
        h = jnp.maximum(
            jnp.dot(t, w2[...], preferred_element_type=jnp.float32)
            + b2[...], 0.0)
        h = h * s2[...] + be2[...]
        outs.append(jnp.where(half == 0, h[:, :_HH], h[:, _HH:]))
    o_ref[...] = jnp.concatenate(outs, axis=1)


def _mlp_tc(zr, xr, eps, w1, b1, s1, be1, w2, b2, s2, be2):
    vec = pl.BlockSpec((1, _H), lambda i: (0, 0))
    mat = pl.BlockSpec((_H, _H), lambda i: (0, 0))
    lo = pl.BlockSpec((_PROWS, 128), lambda i: (i % _PBLK, 0))
    hi = pl.BlockSpec((_PROWS, 128), lambda i: (_PBLK + i % _PBLK, 0))
    return pl.pallas_call(
        _mlp_body,
        grid=(2 * _PBLK,),
        in_specs=[lo, hi, lo, hi,
                  pl.BlockSpec((1, 1), lambda i: (0, 0)),
                  mat, vec, vec, vec, mat, vec, vec, vec],
        out_specs=pl.BlockSpec((_PROWS, 128), lambda i: (i, 0)),
        out_shape=jax.ShapeDtypeStruct((25600, 128), jnp.float32),
    )(zr, zr, xr, xr, eps, w1, b1, s1, be1, w2, b2, s2, be2)


def _mlp1_body(aggp_ref, xs_ref, w1, b1, s1, be1, w2, b2, s2, be2, o_ref):
    a = jnp.sum(aggp_ref[...], axis=0)
    z = xs_ref[...][:, 0] + a
    t = jnp.maximum(z[:, None] * w1[...] + b1[...], 0.0)
    t = t * s1[...] + be1[...]
    h = jnp.maximum(
        jnp.dot(t, w2[...], preferred_element_type=jnp.float32) + b2[...], 0.0)
    _store_half(o_ref, h * s2[...] + be2[...])


def _mlp1_tc(aggp, xs, w1, b1, s1, be1, w2, b2, s2, be2):
    vec = pl.BlockSpec((1, _H), lambda i: (0, 0))
    return pl.pallas_call(
        _mlp1_body,
        grid=(2 * _NBLK,),
        in_specs=[pl.BlockSpec((_NW, _BLK), lambda i: (0, i % _NBLK)),
                  pl.BlockSpec((_BLK, 1), lambda i: (i % _NBLK, 0)),
                  pl.BlockSpec((1, _H), lambda i: (0, 0)),
                  vec, vec, vec,
                  pl.BlockSpec((_H, _H), lambda i: (0, 0)),
                  vec, vec, vec],
        out_specs=pl.BlockSpec((_BLK, _HH), lambda i: (i, 0)),
        out_shape=jax.ShapeDtypeStruct((_N2, _HH), jnp.float32),
    )(aggp, xs, w1, b1, s1, be1, w2, b2, s2, be2)


def _head_body(*refs):
    prefs = refs[:2 * _NLAYERS]
    cnt_ref, w1r, b1, w2, b2, o_ref = refs[2 * _NLAYERS:]
    cnt = jnp.sum(cnt_ref[...], axis=0)[:, 0:1]
    cnt = jnp.maximum(cnt, 1.0)
    w1 = w1r[...]
    acc = jnp.zeros((_B, _H), jnp.float32)
    for l in range(_NLAYERS):
        plo = jnp.sum(prefs[2 * l][...], axis=0)
        phi = jnp.sum(prefs[2 * l + 1][...], axis=0)
        acc = acc + jnp.dot(plo / cnt, w1[l][:_HH],
                            preferred_element_type=jnp.float32)
        acc = acc + jnp.dot(phi / cnt, w1[l][_HH:],
                            preferred_element_type=jnp.float32)
    h = jnp.maximum(acc + b1[...], 0.0)
    logits = jnp.dot(h, w2[...], preferred_element_type=jnp.float32) + b2[...]
    m = jnp.max(logits, axis=-1, keepdims=True)
    lse = m + jnp.log(jnp.sum(jnp.exp(logits - m), axis=-1, keepdims=True))
    o_ref[...] = logits - lse


def _head_tc(pools, cnt, w1r, b1, w2, b2):
    return pl.pallas_call(
        _head_body,
        out_shape=jax.ShapeDtypeStruct((_B, 3), jnp.float32),
    )(*pools, cnt, w1r, b1, w2, b2)


def kernel(x, edge_index, batch, conv1_W1, conv1_b1, conv1_g1, conv1_be1,
           conv1_W2, conv1_b2, conv1_g2, conv1_be2, eps0,
           Ws1, bs1, gs1, bes1, Ws2, bs2, gs2, bes2, epss,
           lin1_W, lin1_b, lin2_W, lin2_b):
    f32 = jnp.float32
    inv = 1.0 / jnp.sqrt(jnp.asarray(1.0 + 1e-5, f32))
    src3 = jnp.pad(edge_index[0], (0, _EP - _E)).reshape(_EG, _G, _K)
    dst3 = jnp.pad(edge_index[1], (0, _EP - _E),
                   constant_values=_NP - 1).reshape(_EG, _G, _K)
    x1 = jnp.pad(x[:, 0], (0, _NP - _N))
    batch2 = jnp.pad(batch, (0, _ND - _N),
                     constant_values=_B).reshape(_DC, 1, _K)

    aggp = _agg1_kernel(x1, src3, dst3).reshape(_NW, _NP)
    xs = ((1.0 + eps0) * x1).reshape(_NP, 1)
    h = _mlp1_tc(
        aggp, xs, conv1_W1, conv1_b1.reshape(1, _H),
        (conv1_g1 * inv).reshape(1, _H), conv1_be1.reshape(1, _H),
        conv1_W2, conv1_b2.reshape(1, _H),
        (conv1_g2 * inv).reshape(1, _H), conv1_be2.reshape(1, _H))
    hs = [h]
    for i in range(_NLAYERS - 1):
        z2 = _gin_agg_kernel(h, src3, dst3)
        h = _mlp_tc(
            z2.reshape(25600, 128), h.reshape(25600, 128),
            epss[i].reshape(1, 1),
            Ws1[i], bs1[i].reshape(1, _H), (gs1[i] * inv).reshape(1, _H),
            bes1[i].reshape(1, _H), Ws2[i], bs2[i].reshape(1, _H),
            (gs2[i] * inv).reshape(1, _H), bes2[i].reshape(1, _H))
        h = h.reshape(_N2, _HH)
        hs.append(h)

    pool_res = _pool_kernel(*hs, batch2)
    pools, cnt = pool_res[:2 * _NLAYERS], pool_res[2 * _NLAYERS]
    return _head_tc(pools, cnt, lin1_W.reshape(_NLAYERS, _H, _H),
                    lin1_b.reshape(1, _H), lin2_W, lin2_b.reshape(1, 3))


# speedup vs baseline: 1.0290x; 1.0290x over previous
import functools

import jax
import jax.numpy as jnp
from jax import lax
from jax.experimental import pallas as pl
from jax.experimental.pallas import tpu as pltpu
from jax.experimental.pallas import tpu_sc as plsc

_N = 50000
_E = 800000
_B = 512
_H = 64
_HH = 32
_NLAYERS = 8
_NC = 2
_NS = 16
_NW = _NC * _NS

_NP = 51200
_RPT = _NP // _NS
_N2 = 2 * _NP

_K = 128
_G = 3
_GE = _G * _K
_EP = 801024
_EG = _EP // _GE

_ND = 50048
_DC = _ND // _K

_mesh = plsc.VectorSubcoreMesh(core_axis_name="c", subcore_axis_name="s")
_sc_params = pltpu.CompilerParams(
    needs_layout_passes=False, use_tc_tiling_on_sc=False)


@functools.partial(
    pl.kernel,
    out_type=jax.ShapeDtypeStruct((_NW * _NP,), jnp.float32),
    mesh=_mesh,
    compiler_params=_sc_params,
    scratch_types=[
        pltpu.VMEM((_NP,), jnp.float32),
        pltpu.VMEM((_NP,), jnp.float32),
        pltpu.VMEM((_G, _K), jnp.int32),
        pltpu.VMEM((_G, _K), jnp.int32),
    ],
)
def _agg1_kernel(x1_hbm, src_hbm, dst_hbm, out_hbm, xv, aggv, srcv, dstv):
    c = lax.axis_index("c")
    s = lax.axis_index("s")
    wid = s * _NC + c
    pltpu.sync_copy(x1_hbm, xv)
    zero16 = jnp.zeros((16,), jnp.float32)

    def _zero(i, carry):
        aggv[pl.ds(i * 16, 16)] = zero16
        return carry

    lax.fori_loop(0, _NP // 16, _zero, 0)

    def _grp(i, carry):
        gg = wid + i * _NW

        @pl.when(gg < _EG)
        def _():
            pltpu.sync_copy(src_hbm.at[gg], srcv)
            pltpu.sync_copy(dst_hbm.at[gg], dstv)
            for j in range(_G):
                for k in range(_K // 16):
                    sidx = srcv[j, pl.ds(k * 16, 16)]
                    didx = dstv[j, pl.ds(k * 16, 16)]
                    vals = plsc.load_gather(xv, [sidx])
                    plsc.addupdate_scatter(aggv, [didx], vals)

        return carry

    lax.fori_loop(0, (_EG + _NW - 1) // _NW, _grp, 0)
    pltpu.sync_copy(aggv, out_hbm.at[pl.ds(wid * _NP, _NP)])


@functools.partial(
    pl.kernel,
    out_type=jax.ShapeDtypeStruct((_N2, _HH), jnp.float32),
    mesh=_mesh,
    compiler_params=_sc_params,
    scratch_types=[
        pltpu.VMEM((2, _GE, _HH), jnp.float32),
        pltpu.VMEM((3, _G, _K), jnp.int32),
        pltpu.VMEM((3, _G, _K), jnp.int32),
        pltpu.VMEM_SHARED((_NP + 8, _HH), jnp.float32),
        pltpu.SemaphoreType.DMA,
        pltpu.SemaphoreType.DMA,
        pltpu.SemaphoreType.DMA,
    ],
)
def _gin_agg_kernel(xlo_hbm, xhi_hbm, src_hbm, dst_hbm, z_hbm,
                    rows, srcv, dstv, agg_sh, gsem, isem, ssem):
    c = lax.axis_index("c")
    s = lax.axis_index("s")
    lbase = s * _RPT
    gbase = c * _NP + s * _RPT

    def _run(base):
        _run_phases(base, src_hbm, dst_hbm, s, rows, srcv, dstv, agg_sh,
                    gsem, isem, ssem)

    @pl.when(c == 0)
    def _():
        _run(xlo_hbm)

    @pl.when(c == 1)
    def _():
        _run(xhi_hbm)

    plsc.subcore_barrier()
    pltpu.sync_copy(agg_sh.at[pl.ds(lbase, _RPT)], z_hbm.at[pl.ds(gbase, _RPT)])


def _run_phases(base, src_hbm, dst_hbm, s, rows, srcv, dstv, agg_sh,
                gsem, isem, ssem):
    lbase = s * _RPT

    pltpu.sync_copy(base.at[pl.ds(lbase, _RPT)], agg_sh.at[pl.ds(lbase, _RPT)])
    plsc.subcore_barrier()

    def _grp(i, carry):
        slot = lax.rem(i, 3)
        nslot = lax.rem(i + 1, 3)
        dslot = lax.rem(i + 1, 3)
        p = lax.rem(i, 2)
        gg = s + i * _NS

        @pl.when(gg < _EG)
        def _():
            @pl.when(i == 0)
            def _():
                pltpu.sync_copy(src_hbm.at[gg], srcv.at[0])
                pltpu.sync_copy(dst_hbm.at[gg], dstv.at[0])

            @pl.when(i > 0)
            def _():
                pltpu.make_async_copy(src_hbm.at[gg], srcv.at[slot],
                                      isem).wait()
                pltpu.make_async_copy(dst_hbm.at[gg], dstv.at[slot],
                                      isem).wait()

            @pl.when(i > 1)
            def _():
                for j in range(_G):
                    pltpu.make_async_copy(
                        rows.at[p, pl.ds(j * _K, _K)],
                        agg_sh.at[dstv.at[dslot, j]], ssem).wait()

            gn = gg + _NS

            @pl.when(gn < _EG)
            def _():
                pltpu.async_copy(src_hbm.at[gn], srcv.at[nslot], isem)
                pltpu.async_copy(dst_hbm.at[gn], dstv.at[nslot], isem)

            gs = [
                pltpu.async_copy(base.at[srcv.at[slot, j]],
                                 rows.at[p, pl.ds(j * _K, _K)], gsem)
                for j in range(_G)
            ]
            for j in range(_G):
                gs[j].wait()
                pltpu.async_copy(rows.at[p, pl.ds(j * _K, _K)],
                                 agg_sh.at[dstv.at[slot, j]], ssem, add=True)

        return carry

    lax.fori_loop(0, (_EG + _NS - 1) // _NS, _grp, 0)

    ni = (_EG - 1 - s) // _NS + 1

    def _drain(d):
        @pl.when(d >= 0)
        def _():
            pd = lax.rem(d, 2)
            sd = lax.rem(d, 3)
            for j in range(_G):
                pltpu.make_async_copy(rows.at[pd, pl.ds(j * _K, _K)],
                                      agg_sh.at[dstv.at[sd, j]], ssem).wait()

    _drain(ni - 2)
    _drain(ni - 1)


_pool_out = tuple(
    jax.ShapeDtypeStruct((_NC, _B, _HH), jnp.float32)
    for _ in range(2 * _NLAYERS + 1)
)
_PSH = 528


@functools.partial(
    pl.kernel,
    out_type=_pool_out,
    mesh=_mesh,
    compiler_params=_sc_params,
    scratch_types=[
        pltpu.VMEM((2 * _NLAYERS, _K, _HH), jnp.float32),
        pltpu.VMEM((_K, _HH), jnp.float32),
        pltpu.VMEM((1, _K), jnp.int32),
        pltpu.SemaphoreType.DMA,
    ]
    + [pltpu.VMEM_SHARED((_PSH, _HH), jnp.float32)
       for _ in range(2 * _NLAYERS + 1)],
)
def _pool_kernel(*args):
    hs = args[:2 * _NLAYERS]
    batch_hbm = args[2 * _NLAYERS]
    rest = args[2 * _NLAYERS + 1:]
    outs = rest[:2 * _NLAYERS + 1]
    rows16, ones, bidx, sem = rest[2 * _NLAYERS + 1:2 * _NLAYERS + 5]
    shs = rest[2 * _NLAYERS + 5:]
    c = lax.axis_index("c")
    s = lax.axis_index("s")
    wid = s * _NC + c
    zero16 = jnp.zeros((16,), jnp.float32)
    one16 = jnp.ones((16,), jnp.float32)

    def _init(i, carry):
        for k in range(_HH // 16):
            ones[i, pl.ds(k * 16, 16)] = one16
            rows16[0, i, pl.ds(k * 16, 16)] = zero16
        return carry

    lax.fori_loop(0, _K, _init, 0)
    for sh in shs:
        pltpu.sync_copy(rows16.at[0, pl.ds(0, 33)], sh.at[pl.ds(s * 33, 33)])
    plsc.subcore_barrier()

    def _chunk(i, carry):
        cc = wid + i * _NW

        @pl.when(cc < _DC)
        def _():
            pltpu.sync_copy(batch_hbm.at[cc], bidx)
            cps = []
            for q in range(2 * _NLAYERS):
                cps.append(pltpu.async_copy(
                    hs[q].at[pl.ds(cc * _K, _K)], rows16.at[q], sem))
            for q, cp in enumerate(cps):
                cp.wait()
                pltpu.sync_copy(rows16.at[q], shs[q].at[bidx.at[0]], add=True)
            pltpu.sync_copy(ones, shs[2 * _NLAYERS].at[bidx.at[0]], add=True)

        return carry

    lax.fori_loop(0, (_DC + _NW - 1) // _NW, _chunk, 0)
    plsc.subcore_barrier()
    for q in range(2 * _NLAYERS + 1):
        pltpu.sync_copy(shs[q].at[pl.ds(s * 32, 32)],
                        outs[q].at[c, pl.ds(s * 32, 32)])


_BLK = 2048
_NBLK = _NP // _BLK


_PBLK = 8
_PROWS = 12800 // _PBLK


def _mlp_body(zl_ref, zh_ref, xl_ref, xh_ref, eps_ref,
              w1, b1, s1, be1, w2, b2, s2, be2, olo_ref, ohi_ref):
    e = eps_ref[0, 0]
    zl = zl_ref[...] + e * xl_ref[...]
    zh = zh_ref[...] + e * xh_ref[...]
    los, his = [], []
    for q in range(4):
        z = jnp.concatenate(
            [zl[:, q * _HH:(q + 1) * _HH], zh[:, q * _HH:(q + 1) * _HH]],
            axis=1)
        t = jnp.maximum(
            jnp.dot(z, w1[...], preferred_element_type=jnp.float32)
            + b1[...], 0.0)
        t = t * s1[...] + be1[...]
        h = jnp.maximum(
            jnp.dot(t, w2[...], preferred_element_type=jnp.float32)
            + b2[...], 0.0)
        h = h * s2[...] + be2[...]
        los.append(h[:, :_HH])
        his.append(h[:, _HH:])
    olo_ref[...] = jnp.concatenate(los, axis=1)
    ohi_ref[...] = jnp.concatenate(his, axis=1)


def _mlp_tc(zr, xlo, xhi, eps, w1, b1, s1, be1, w2, b2, s2, be2):
    vec = pl.BlockSpec((1, _H), lambda i: (0, 0))
    mat = pl.BlockSpec((_H, _H), lambda i: (0, 0))
    blk = pl.BlockSpec((_PROWS, 128), lambda i: (i, 0))
    zhi = pl.BlockSpec((_PROWS, 128), lambda i: (_PBLK + i, 0))
    half = jax.ShapeDtypeStruct((12800, 128), jnp.float32)
    return pl.pallas_call(
        _mlp_body,
        grid=(_PBLK,),
        in_specs=[blk, zhi, blk, blk,
                  pl.BlockSpec((1, 1), lambda i: (0, 0)),
                  mat, vec, vec, vec, mat, vec, vec, vec],
        out_specs=[blk, blk],
        out_shape=(half, half),
    )(zr, zr, xlo, xhi, eps, w1, b1, s1, be1, w2, b2, s2, be2)


_B1 = _NP // _NBLK // 4


def _mlp1_body(aggp_ref, xs_ref, w1, b1, s1, be1, w2, b2, s2, be2,
               olo_ref, ohi_ref):
    a = jnp.sum(aggp_ref[...], axis=0)
    z4 = xs_ref[...] + a
    los, his = [], []
    for q in range(4):
        z = z4[:, q]
        t = jnp.maximum(z[:, None] * w1[...] + b1[...], 0.0)
        t = t * s1[...] + be1[...]
        h = jnp.maximum(
            jnp.dot(t, w2[...], preferred_element_type=jnp.float32)
            + b2[...], 0.0)
        h = h * s2[...] + be2[...]
        los.append(h[:, :_HH])
        his.append(h[:, _HH:])
    olo_ref[...] = jnp.concatenate(los, axis=1)
    ohi_ref[...] = jnp.concatenate(his, axis=1)


def _mlp1_tc(aggp4, xs4, w1, b1, s1, be1, w2, b2, s2, be2):
    vec = pl.BlockSpec((1, _H), lambda i: (0, 0))
    half = jax.ShapeDtypeStruct((12800, 128), jnp.float32)
    return pl.pallas_call(
        _mlp1_body,
        grid=(_NBLK,),
        in_specs=[pl.BlockSpec((_NW, _B1, 4), lambda i: (0, i, 0)),
                  pl.BlockSpec((_B1, 4), lambda i: (i, 0)),
                  pl.BlockSpec((1, _H), lambda i: (0, 0)),
                  vec, vec, vec,
                  pl.BlockSpec((_H, _H), lambda i: (0, 0)),
                  vec, vec, vec],
        out_specs=[pl.BlockSpec((_B1, 128), lambda i: (i, 0)),
                   pl.BlockSpec((_B1, 128), lambda i: (i, 0))],
        out_shape=(half, half),
    )(aggp4, xs4, w1, b1, s1, be1, w2, b2, s2, be2)


def _head_body(*refs):
    prefs = refs[:2 * _NLAYERS]
    cnt_ref, w1r, b1, w2, b2, o_ref = refs[2 * _NLAYERS:]
    cnt = jnp.sum(cnt_ref[...], axis=0)[:, 0:1]
    cnt = jnp.maximum(cnt, 1.0)
    w1 = w1r[...]
    acc = jnp.zeros((_B, _H), jnp.float32)
    for l in range(_NLAYERS):
        plo = jnp.sum(prefs[2 * l][...], axis=0)
        phi = jnp.sum(prefs[2 * l + 1][...], axis=0)
        acc = acc + jnp.dot(plo / cnt, w1[l][:_HH],
                            preferred_element_type=jnp.float32)
        acc = acc + jnp.dot(phi / cnt, w1[l][_HH:],
                            preferred_element_type=jnp.float32)
    h = jnp.maximum(acc + b1[...], 0.0)
    logits = jnp.dot(h, w2[...], preferred_element_type=jnp.float32) + b2[...]
    m = jnp.max(logits, axis=-1, keepdims=True)
    lse = m + jnp.log(jnp.sum(jnp.exp(logits - m), axis=-1, keepdims=True))
    o_ref[...] = logits - lse


def _head_tc(pools, cnt, w1r, b1, w2, b2):
    return pl.pallas_call(
        _head_body,
        out_shape=jax.ShapeDtypeStruct((_B, 3), jnp.float32),
    )(*pools, cnt, w1r, b1, w2, b2)


def kernel(x, edge_index, batch, conv1_W1, conv1_b1, conv1_g1, conv1_be1,
           conv1_W2, conv1_b2, conv1_g2, conv1_be2, eps0,
           Ws1, bs1, gs1, bes1, Ws2, bs2, gs2, bes2, epss,
           lin1_W, lin1_b, lin2_W, lin2_b):
    f32 = jnp.float32
    inv = 1.0 / jnp.sqrt(jnp.asarray(1.0 + 1e-5, f32))
    src3 = jnp.pad(edge_index[0], (0, _EP - _E)).reshape(_EG, _G, _K)
    dst3 = jnp.pad(edge_index[1], (0, _EP - _E),
                   constant_values=_NP - 1).reshape(_EG, _G, _K)
    x1 = jnp.pad(x[:, 0], (0, _NP - _N))
    batch2 = jnp.pad(batch, (0, _ND - _N),
                     constant_values=_B).reshape(_DC, 1, _K)

    aggp4 = _agg1_kernel(x1, src3, dst3).reshape(_NW, _NP // 4, 4)
    xs4 = ((1.0 + eps0) * x1).reshape(_NP // 4, 4)
    hlo, hhi = _mlp1_tc(
        aggp4, xs4, conv1_W1, conv1_b1.reshape(1, _H),
        (conv1_g1 * inv).reshape(1, _H), conv1_be1.reshape(1, _H),
        conv1_W2, conv1_b2.reshape(1, _H),
        (conv1_g2 * inv).reshape(1, _H), conv1_be2.reshape(1, _H))
    hs = [hlo.reshape(_NP, _HH), hhi.reshape(_NP, _HH)]
    for i in range(_NLAYERS - 1):
        z2 = _gin_agg_kernel(hs[-2], hs[-1], src3, dst3)
        hlo, hhi = _mlp_tc(
            z2.reshape(25600, 128), hlo, hhi, epss[i].reshape(1, 1),
            Ws1[i], bs1[i].reshape(1, _H), (gs1[i] * inv).reshape(1, _H),
            bes1[i].reshape(1, _H), Ws2[i], bs2[i].reshape(1, _H),
            (gs2[i] * inv).reshape(1, _H), bes2[i].reshape(1, _H))
        hs.extend([hlo.reshape(_NP, _HH), hhi.reshape(_NP, _HH)])

    pool_res = _pool_kernel(*hs, batch2)
    pools, cnt = pool_res[:2 * _NLAYERS], pool_res[2 * _NLAYERS]
    return _head_tc(pools, cnt, lin1_W.reshape(_NLAYERS, _H, _H),
                    lin1_b.reshape(1, _H), lin2_W, lin2_b.reshape(1, 3))
